# baseline (speedup 1.0000x reference)
import jax
import jax.numpy as jnp
from jax import lax
from jax.experimental import pallas as pl
from jax.experimental.pallas import tpu as pltpu

B, N, M = 8, 2048, 2048
K = 5


def _chamfer_body(l_ref, r_ref, out_ref):
    acc = jnp.float32(0.0)
    for bi in range(B):
        l = l_ref[bi]
        r = r_ref[bi]
        d = lax.dot_general(l, r, (((1,), (1,)), ((), ())),
                            preferred_element_type=jnp.float32)
        s1 = jnp.sum(jnp.maximum(jnp.min(d, axis=1), 0.0))
        s2 = jnp.sum(jnp.maximum(jnp.min(d, axis=0), 0.0))
        acc = acc + s1 + s2
    out_ref[...] = jnp.reshape(acc, (1, 1))


def kernel(array1, array2):
    a2 = jnp.sum(array1 * array1, axis=2, keepdims=True)
    b2 = jnp.sum(array2 * array2, axis=2, keepdims=True)
    ones_a = jnp.ones_like(a2)
    l_aug = jnp.concatenate([array1, a2, ones_a], axis=2)
    r_aug = jnp.concatenate([-2.0 * array2, ones_a, b2], axis=2)
    out = pl.pallas_call(
        _chamfer_body,
        out_shape=jax.ShapeDtypeStruct((1, 1), jnp.float32),
    )(l_aug, r_aug)
    return out[0, 0] * (1.0 / (B * N))


# speedup vs baseline: 1.5021x; 1.5021x over previous
import jax
import jax.numpy as jnp
from jax import lax
from jax.experimental import pallas as pl
from jax.experimental.pallas import tpu as pltpu

B, N, M = 8, 2048, 2048
K = 5
CB = 1
MB = M // CB


def _chamfer_body(l_ref, r_ref, out_ref):
    acc = jnp.float32(0.0)
    for bi in range(B):
        l = l_ref[bi]
        r = r_ref[bi]
        s2 = jnp.float32(0.0)
        rowacc = None
        for cb in range(CB):
            rblk = r[:, cb * MB:(cb + 1) * MB]
            dblk = lax.dot_general(l, rblk, (((0,), (0,)), ((), ())),
                                   preferred_element_type=jnp.float32)
            rm = jnp.min(dblk, axis=1)
            rowacc = rm if rowacc is None else jnp.minimum(rowacc, rm)
            s2 = s2 + jnp.sum(jnp.maximum(jnp.min(dblk, axis=0), 0.0))
        s1 = jnp.sum(jnp.maximum(rowacc, 0.0))
        acc = acc + s1 + s2
    out_ref[...] = jnp.reshape(acc, (1, 1))


def kernel(array1, array2):
    a_t = jnp.transpose(array1, (0, 2, 1))
    b_t = jnp.transpose(array2, (0, 2, 1))
    a2 = jnp.sum(a_t * a_t, axis=1, keepdims=True)
    b2 = jnp.sum(b_t * b_t, axis=1, keepdims=True)
    ones_a = jnp.ones_like(a2)
    l_aug = jnp.concatenate([a_t, a2, ones_a], axis=1)
    r_aug = jnp.concatenate([-2.0 * b_t, ones_a, b2], axis=1)
    out = pl.pallas_call(
        _chamfer_body,
        out_shape=jax.ShapeDtypeStruct((1, 1), jnp.float32),
    )(l_aug, r_aug)
    return out[0, 0] * (1.0 / (B * N))
